# baseline (speedup 1.0000x reference)
import functools

import jax
import jax.numpy as jnp
from jax import lax
from jax.experimental import pallas as pl
from jax.experimental.pallas import tpu as pltpu
from jax.experimental.pallas import tpu_sc as plsc

N = 18432
K = 8192
D = 256
MB = 768
NR = N // MB
KB = 2048
NC = K // KB
COST = 0.25
KHI = 64
KLO = 128

_I32_BIG = 2 ** 30


def _argmin_body(zn_ref, z_ref, et2_ref, en_ref, idx_ref, loss_ref, perp_ref,
                 nact_ref, cnt_ref, acc_ref):
    i = pl.program_id(0)
    z = z_ref[...]
    zn = zn_ref[...]
    run_d = None
    run_c = None
    for c in range(NC):
        dot2 = jnp.dot(z, et2_ref[:, c * KB:(c + 1) * KB],
                       preferred_element_type=jnp.float32)
        s = (zn - dot2) + en_ref[:, c * KB:(c + 1) * KB]
        if c == 0:
            run_d = s
            run_c = jnp.zeros((MB, KB), jnp.int32)
        else:
            m = s < run_d
            run_d = jnp.minimum(s, run_d)
            run_c = jnp.where(m, c, run_c)
    smin = jnp.min(run_d, axis=1, keepdims=True)
    cmin = jnp.sqrt(jnp.maximum(smin, 0.0))
    run_d = jnp.sqrt(jnp.maximum(run_d, 0.0))
    iota = lax.broadcasted_iota(jnp.int32, (MB, KB), 1)
    gidx = run_c * KB + iota
    bidx = jnp.min(jnp.where(run_d == cmin, gidx, _I32_BIG),
                   axis=1, keepdims=True)
    idx_ref[...] = bidx

    lsum = jnp.sum(jnp.maximum(smin, 0.0)).reshape(1, 1)

    hi1 = (lax.broadcasted_iota(jnp.int32, (MB, KHI), 1)
           == (bidx >> 7)).astype(jnp.float32)
    lo1 = (lax.broadcasted_iota(jnp.int32, (MB, KLO), 1)
           == (bidx & (KLO - 1))).astype(jnp.float32)
    hist = jax.lax.dot_general(hi1, lo1, (((0,), (0,)), ((), ())),
                               preferred_element_type=jnp.float32)

    @pl.when(i == 0)
    def _():
        acc_ref[...] = lsum
        cnt_ref[...] = hist

    @pl.when(i > 0)
    def _():
        acc_ref[...] = acc_ref[...] + lsum
        cnt_ref[...] = cnt_ref[...] + hist

    @pl.when(i == NR - 1)
    def _():
        loss_ref[...] = COST * (acc_ref[...] / jnp.float32(N * D))
        avg = cnt_ref[...] / jnp.float32(N)
        ent = jnp.sum(avg * jnp.log(avg + 1e-10)).reshape(1, 1)
        perp_ref[...] = jnp.exp(-ent)
        nact_ref[...] = jnp.sum((avg > 0.001).astype(jnp.int32)).reshape(1, 1)


def _argmin_call(zn, z_e, et2, en):
    return pl.pallas_call(
        _argmin_body,
        grid=(NR,),
        in_specs=[
            pl.BlockSpec((MB, 1), lambda i: (i, 0)),
            pl.BlockSpec((MB, D), lambda i: (i, 0)),
            pl.BlockSpec((D, K), lambda i: (0, 0)),
            pl.BlockSpec((1, K), lambda i: (0, 0)),
        ],
        out_specs=[
            pl.BlockSpec((MB, 1), lambda i: (i, 0)),
            pl.BlockSpec((1, 1), lambda i: (0, 0)),
            pl.BlockSpec((1, 1), lambda i: (0, 0)),
            pl.BlockSpec((1, 1), lambda i: (0, 0)),
        ],
        out_shape=[
            jax.ShapeDtypeStruct((N, 1), jnp.int32),
            jax.ShapeDtypeStruct((1, 1), jnp.float32),
            jax.ShapeDtypeStruct((1, 1), jnp.float32),
            jax.ShapeDtypeStruct((1, 1), jnp.int32),
        ],
        scratch_shapes=[
            pltpu.VMEM((KHI, KLO), jnp.float32),
            pltpu.VMEM((1, 1), jnp.float32),
        ],
    )(zn, z_e, et2, en)


_SC_INFO = plsc.get_sparse_core_info()
_NW = _SC_INFO.num_cores * _SC_INFO.num_subcores
_BW = N // _NW
_CH = 192
_NCH = _BW // _CH


@functools.partial(
    pl.kernel,
    mesh=plsc.VectorSubcoreMesh(core_axis_name="c", subcore_axis_name="s"),
    out_type=jax.ShapeDtypeStruct((N, D), jnp.float32),
    scratch_types=[
        pltpu.VMEM((_BW,), jnp.int32),
        pltpu.VMEM((_CH, D), jnp.float32),
        pltpu.VMEM((_CH, D), jnp.float32),
        pltpu.SemaphoreType.DMA,
        pltpu.SemaphoreType.DMA,
        pltpu.SemaphoreType.DMA,
        pltpu.SemaphoreType.DMA,
    ],
)
def _gather_rows(emb_hbm, idx_hbm, out_hbm, idx_v, b0, b1, g0, g1, w0, w1):
    wid = lax.axis_index("s") * _SC_INFO.num_cores + lax.axis_index("c")
    base = wid * _BW
    pltpu.sync_copy(idx_hbm.at[pl.ds(base, _BW)], idx_v)
    bufs, gsems, wsems = (b0, b1), (g0, g1), (w0, w1)
    gathers = [None] * _NCH
    writes = [None] * _NCH
    gathers[0] = pltpu.async_copy(
        emb_hbm.at[idx_v.at[pl.ds(0, _CH)]], bufs[0], gsems[0])
    for j in range(_NCH):
        nxt = j + 1
        if nxt < _NCH:
            if nxt - 2 >= 0:
                writes[nxt - 2].wait()
            gathers[nxt] = pltpu.async_copy(
                emb_hbm.at[idx_v.at[pl.ds(nxt * _CH, _CH)]],
                bufs[nxt % 2], gsems[nxt % 2])
        gathers[j].wait()
        writes[j] = pltpu.async_copy(
            bufs[j % 2], out_hbm.at[pl.ds(base + j * _CH, _CH)], wsems[j % 2])
    writes[_NCH - 2].wait()
    writes[_NCH - 1].wait()


def _zst_body(z_ref, q_ref, zst_ref):
    z = z_ref[...]
    zst_ref[...] = z + (q_ref[...] - z)


def _zst_call(z_e, z_q):
    return pl.pallas_call(
        _zst_body,
        grid=(NR,),
        in_specs=[
            pl.BlockSpec((MB, D), lambda i: (i, 0)),
            pl.BlockSpec((MB, D), lambda i: (i, 0)),
        ],
        out_specs=pl.BlockSpec((MB, D), lambda i: (i, 0)),
        out_shape=jax.ShapeDtypeStruct((N, D), jnp.float32),
    )(z_e, z_q)


def kernel(z_e, embeddings):
    zn = jnp.sum(z_e ** 2, axis=1, keepdims=True)
    en = jnp.sum(embeddings ** 2, axis=1)[None, :]
    et2 = embeddings.T * 2.0
    idx2, loss, perp, nact = _argmin_call(zn, z_e, et2, en)
    indices = idx2.reshape(N)
    z_q = _gather_rows(embeddings, indices)
    zst = _zst_call(z_e, z_q)
    return (zst, indices, loss[0, 0], perp[0, 0], nact[0, 0])


# speedup vs baseline: 1.1725x; 1.1725x over previous
import functools

import jax
import jax.numpy as jnp
from jax import lax
from jax.experimental import pallas as pl
from jax.experimental.pallas import tpu as pltpu
from jax.experimental.pallas import tpu_sc as plsc

N = 18432
K = 8192
D = 256
MB = 768
NR = N // MB
KB = 512
NC = K // KB
COST = 0.25
KHI = 64
KLO = 128

_I32_BIG = 2 ** 30


def _argmin_body(zn_ref, z_ref, et2_ref, en_ref, idx_ref, loss_ref, perp_ref,
                 nact_ref, cnt_ref, acc_ref):
    i = pl.program_id(0)
    z = z_ref[...]
    zn = zn_ref[...]
    run_d = None
    run_c = None
    for c in range(NC):
        dot2 = jnp.dot(z, et2_ref[:, c * KB:(c + 1) * KB],
                       preferred_element_type=jnp.float32)
        s = (zn - dot2) + en_ref[:, c * KB:(c + 1) * KB]
        if c == 0:
            run_d = s
            run_c = jnp.zeros((MB, KB), jnp.int32)
        else:
            m = s < run_d
            run_d = jnp.minimum(s, run_d)
            run_c = jnp.where(m, c, run_c)
    smin = jnp.min(run_d, axis=1, keepdims=True)
    cmin = jnp.sqrt(jnp.maximum(smin, 0.0))
    run_d = jnp.sqrt(jnp.maximum(run_d, 0.0))
    iota = lax.broadcasted_iota(jnp.int32, (MB, KB), 1)
    gidx = run_c * KB + iota
    bidx = jnp.min(jnp.where(run_d == cmin, gidx, _I32_BIG),
                   axis=1, keepdims=True)
    idx_ref[...] = bidx

    lsum = jnp.sum(jnp.maximum(smin, 0.0)).reshape(1, 1)

    hi1 = (lax.broadcasted_iota(jnp.int32, (MB, KHI), 1)
           == (bidx >> 7)).astype(jnp.float32)
    lo1 = (lax.broadcasted_iota(jnp.int32, (MB, KLO), 1)
           == (bidx & (KLO - 1))).astype(jnp.float32)
    hist = jax.lax.dot_general(hi1, lo1, (((0,), (0,)), ((), ())),
                               preferred_element_type=jnp.float32)

    @pl.when(i == 0)
    def _():
        acc_ref[...] = lsum
        cnt_ref[...] = hist

    @pl.when(i > 0)
    def _():
        acc_ref[...] = acc_ref[...] + lsum
        cnt_ref[...] = cnt_ref[...] + hist

    @pl.when(i == NR - 1)
    def _():
        loss_ref[...] = COST * (acc_ref[...] / jnp.float32(N * D))
        avg = cnt_ref[...] / jnp.float32(N)
        ent = jnp.sum(avg * jnp.log(avg + 1e-10)).reshape(1, 1)
        perp_ref[...] = jnp.exp(-ent)
        nact_ref[...] = jnp.sum((avg > 0.001).astype(jnp.int32)).reshape(1, 1)


def _argmin_call(zn, z_e, et2, en):
    return pl.pallas_call(
        _argmin_body,
        grid=(NR,),
        in_specs=[
            pl.BlockSpec((MB, 1), lambda i: (i, 0)),
            pl.BlockSpec((MB, D), lambda i: (i, 0)),
            pl.BlockSpec((D, K), lambda i: (0, 0)),
            pl.BlockSpec((1, K), lambda i: (0, 0)),
        ],
        out_specs=[
            pl.BlockSpec((MB, 1), lambda i: (i, 0)),
            pl.BlockSpec((1, 1), lambda i: (0, 0)),
            pl.BlockSpec((1, 1), lambda i: (0, 0)),
            pl.BlockSpec((1, 1), lambda i: (0, 0)),
        ],
        out_shape=[
            jax.ShapeDtypeStruct((N, 1), jnp.int32),
            jax.ShapeDtypeStruct((1, 1), jnp.float32),
            jax.ShapeDtypeStruct((1, 1), jnp.float32),
            jax.ShapeDtypeStruct((1, 1), jnp.int32),
        ],
        scratch_shapes=[
            pltpu.VMEM((KHI, KLO), jnp.float32),
            pltpu.VMEM((1, 1), jnp.float32),
        ],
    )(zn, z_e, et2, en)


_SC_INFO = plsc.get_sparse_core_info()
_NW = _SC_INFO.num_cores * _SC_INFO.num_subcores
_BW = N // _NW
_CH = 192
_NCH = _BW // _CH


@functools.partial(
    pl.kernel,
    mesh=plsc.VectorSubcoreMesh(core_axis_name="c", subcore_axis_name="s"),
    out_type=jax.ShapeDtypeStruct((N, D), jnp.float32),
    scratch_types=[
        pltpu.VMEM((_BW,), jnp.int32),
        pltpu.VMEM((_CH, D), jnp.float32),
        pltpu.VMEM((_CH, D), jnp.float32),
        pltpu.SemaphoreType.DMA,
        pltpu.SemaphoreType.DMA,
        pltpu.SemaphoreType.DMA,
        pltpu.SemaphoreType.DMA,
    ],
)
def _gather_rows(emb_hbm, idx_hbm, out_hbm, idx_v, b0, b1, g0, g1, w0, w1):
    wid = lax.axis_index("s") * _SC_INFO.num_cores + lax.axis_index("c")
    base = wid * _BW
    pltpu.sync_copy(idx_hbm.at[pl.ds(base, _BW)], idx_v)
    bufs, gsems, wsems = (b0, b1), (g0, g1), (w0, w1)
    gathers = [None] * _NCH
    writes = [None] * _NCH
    gathers[0] = pltpu.async_copy(
        emb_hbm.at[idx_v.at[pl.ds(0, _CH)]], bufs[0], gsems[0])
    for j in range(_NCH):
        nxt = j + 1
        if nxt < _NCH:
            if nxt - 2 >= 0:
                writes[nxt - 2].wait()
            gathers[nxt] = pltpu.async_copy(
                emb_hbm.at[idx_v.at[pl.ds(nxt * _CH, _CH)]],
                bufs[nxt % 2], gsems[nxt % 2])
        gathers[j].wait()
        writes[j] = pltpu.async_copy(
            bufs[j % 2], out_hbm.at[pl.ds(base + j * _CH, _CH)], wsems[j % 2])
    writes[_NCH - 2].wait()
    writes[_NCH - 1].wait()


def _zst_body(z_ref, q_ref, zst_ref):
    z = z_ref[...]
    zst_ref[...] = z + (q_ref[...] - z)


def _zst_call(z_e, z_q):
    return pl.pallas_call(
        _zst_body,
        grid=(NR,),
        in_specs=[
            pl.BlockSpec((MB, D), lambda i: (i, 0)),
            pl.BlockSpec((MB, D), lambda i: (i, 0)),
        ],
        out_specs=pl.BlockSpec((MB, D), lambda i: (i, 0)),
        out_shape=jax.ShapeDtypeStruct((N, D), jnp.float32),
    )(z_e, z_q)


def kernel(z_e, embeddings):
    zn = jnp.sum(z_e ** 2, axis=1, keepdims=True)
    en = jnp.sum(embeddings ** 2, axis=1)[None, :]
    et2 = embeddings.T * 2.0
    idx2, loss, perp, nact = _argmin_call(zn, z_e, et2, en)
    indices = idx2.reshape(N)
    z_q = _gather_rows(embeddings, indices)
    zst = _zst_call(z_e, z_q)
    return (zst, indices, loss[0, 0], perp[0, 0], nact[0, 0])


# speedup vs baseline: 1.2787x; 1.0906x over previous
import functools

import jax
import jax.numpy as jnp
from jax import lax
from jax.experimental import pallas as pl
from jax.experimental.pallas import tpu as pltpu
from jax.experimental.pallas import tpu_sc as plsc

N = 18432
K = 8192
D = 256
MB = 768
NR = N // MB
KB = 256
NC = K // KB
COST = 0.25
KHI = 64
KLO = 128

_I32_BIG = 2 ** 30


def _argmin_body(zn_ref, z_ref, et2_ref, en_ref, idx_ref, loss_ref, perp_ref,
                 nact_ref, cnt_ref, acc_ref):
    i = pl.program_id(0)
    z = z_ref[...]
    zn = zn_ref[...]
    run_d = None
    run_c = None
    for c in range(NC):
        dot2 = jnp.dot(z, et2_ref[:, c * KB:(c + 1) * KB],
                       preferred_element_type=jnp.float32)
        s = (zn - dot2) + en_ref[:, c * KB:(c + 1) * KB]
        if c == 0:
            run_d = s
            run_c = jnp.zeros((MB, KB), jnp.int32)
        else:
            m = s < run_d
            run_d = jnp.minimum(s, run_d)
            run_c = jnp.where(m, c, run_c)
    smin = jnp.min(run_d, axis=1, keepdims=True)
    cmin = jnp.sqrt(jnp.maximum(smin, 0.0))
    run_d = jnp.sqrt(jnp.maximum(run_d, 0.0))
    iota = lax.broadcasted_iota(jnp.int32, (MB, KB), 1)
    gidx = run_c * KB + iota
    bidx = jnp.min(jnp.where(run_d == cmin, gidx, _I32_BIG),
                   axis=1, keepdims=True)
    idx_ref[...] = bidx

    lsum = jnp.sum(jnp.maximum(smin, 0.0)).reshape(1, 1)

    hi1 = (lax.broadcasted_iota(jnp.int32, (MB, KHI), 1)
           == (bidx >> 7)).astype(jnp.float32)
    lo1 = (lax.broadcasted_iota(jnp.int32, (MB, KLO), 1)
           == (bidx & (KLO - 1))).astype(jnp.float32)
    hist = jax.lax.dot_general(hi1, lo1, (((0,), (0,)), ((), ())),
                               preferred_element_type=jnp.float32)

    @pl.when(i == 0)
    def _():
        acc_ref[...] = lsum
        cnt_ref[...] = hist

    @pl.when(i > 0)
    def _():
        acc_ref[...] = acc_ref[...] + lsum
        cnt_ref[...] = cnt_ref[...] + hist

    @pl.when(i == NR - 1)
    def _():
        loss_ref[...] = COST * (acc_ref[...] / jnp.float32(N * D))
        avg = cnt_ref[...] / jnp.float32(N)
        ent = jnp.sum(avg * jnp.log(avg + 1e-10)).reshape(1, 1)
        perp_ref[...] = jnp.exp(-ent)
        nact_ref[...] = jnp.sum((avg > 0.001).astype(jnp.int32)).reshape(1, 1)


def _argmin_call(zn, z_e, et2, en):
    return pl.pallas_call(
        _argmin_body,
        grid=(NR,),
        in_specs=[
            pl.BlockSpec((MB, 1), lambda i: (i, 0)),
            pl.BlockSpec((MB, D), lambda i: (i, 0)),
            pl.BlockSpec((D, K), lambda i: (0, 0)),
            pl.BlockSpec((1, K), lambda i: (0, 0)),
        ],
        out_specs=[
            pl.BlockSpec((MB, 1), lambda i: (i, 0)),
            pl.BlockSpec((1, 1), lambda i: (0, 0)),
            pl.BlockSpec((1, 1), lambda i: (0, 0)),
            pl.BlockSpec((1, 1), lambda i: (0, 0)),
        ],
        out_shape=[
            jax.ShapeDtypeStruct((N, 1), jnp.int32),
            jax.ShapeDtypeStruct((1, 1), jnp.float32),
            jax.ShapeDtypeStruct((1, 1), jnp.float32),
            jax.ShapeDtypeStruct((1, 1), jnp.int32),
        ],
        scratch_shapes=[
            pltpu.VMEM((KHI, KLO), jnp.float32),
            pltpu.VMEM((1, 1), jnp.float32),
        ],
    )(zn, z_e, et2, en)


_SC_INFO = plsc.get_sparse_core_info()
_NW = _SC_INFO.num_cores * _SC_INFO.num_subcores
_BW = N // _NW
_CH = 192
_NCH = _BW // _CH


@functools.partial(
    pl.kernel,
    mesh=plsc.VectorSubcoreMesh(core_axis_name="c", subcore_axis_name="s"),
    out_type=jax.ShapeDtypeStruct((N, D), jnp.float32),
    scratch_types=[
        pltpu.VMEM((_BW,), jnp.int32),
        pltpu.VMEM((_CH, D), jnp.float32),
        pltpu.VMEM((_CH, D), jnp.float32),
        pltpu.SemaphoreType.DMA,
        pltpu.SemaphoreType.DMA,
        pltpu.SemaphoreType.DMA,
        pltpu.SemaphoreType.DMA,
    ],
)
def _gather_rows(emb_hbm, idx_hbm, out_hbm, idx_v, b0, b1, g0, g1, w0, w1):
    wid = lax.axis_index("s") * _SC_INFO.num_cores + lax.axis_index("c")
    base = wid * _BW
    pltpu.sync_copy(idx_hbm.at[pl.ds(base, _BW)], idx_v)
    bufs, gsems, wsems = (b0, b1), (g0, g1), (w0, w1)
    gathers = [None] * _NCH
    writes = [None] * _NCH
    gathers[0] = pltpu.async_copy(
        emb_hbm.at[idx_v.at[pl.ds(0, _CH)]], bufs[0], gsems[0])
    for j in range(_NCH):
        nxt = j + 1
        if nxt < _NCH:
            if nxt - 2 >= 0:
                writes[nxt - 2].wait()
            gathers[nxt] = pltpu.async_copy(
                emb_hbm.at[idx_v.at[pl.ds(nxt * _CH, _CH)]],
                bufs[nxt % 2], gsems[nxt % 2])
        gathers[j].wait()
        writes[j] = pltpu.async_copy(
            bufs[j % 2], out_hbm.at[pl.ds(base + j * _CH, _CH)], wsems[j % 2])
    writes[_NCH - 2].wait()
    writes[_NCH - 1].wait()


def _zst_body(z_ref, q_ref, zst_ref):
    z = z_ref[...]
    zst_ref[...] = z + (q_ref[...] - z)


def _zst_call(z_e, z_q):
    return pl.pallas_call(
        _zst_body,
        grid=(NR,),
        in_specs=[
            pl.BlockSpec((MB, D), lambda i: (i, 0)),
            pl.BlockSpec((MB, D), lambda i: (i, 0)),
        ],
        out_specs=pl.BlockSpec((MB, D), lambda i: (i, 0)),
        out_shape=jax.ShapeDtypeStruct((N, D), jnp.float32),
    )(z_e, z_q)


def kernel(z_e, embeddings):
    zn = jnp.sum(z_e ** 2, axis=1, keepdims=True)
    en = jnp.sum(embeddings ** 2, axis=1)[None, :]
    et2 = embeddings.T * 2.0
    idx2, loss, perp, nact = _argmin_call(zn, z_e, et2, en)
    indices = idx2.reshape(N)
    z_q = _gather_rows(embeddings, indices)
    zst = _zst_call(z_e, z_q)
    return (zst, indices, loss[0, 0], perp[0, 0], nact[0, 0])
